# baseline (speedup 1.0000x reference)
import functools

import jax
import jax.numpy as jnp
from jax import lax
from jax.experimental import pallas as pl
from jax.experimental.pallas import tpu as pltpu
from jax.experimental.pallas import tpu_sc as plsc

_G = 512
_F = 128
_ND = 2 * _F
_NC = 2
_NS = 16
_NW = _NC * _NS
_SPW = _G // _NW
_CHUNK = 256
_LANES = 16
_WIN = 128
_NB = _SPW + 1


def _build_pool(V, interpret=False):
    mesh = plsc.VectorSubcoreMesh(core_axis_name="c", subcore_axis_name="s",
                                  num_cores=_NC, num_subcores=_NS)

    @functools.partial(
        pl.kernel,
        out_type=jax.ShapeDtypeStruct((_G, _ND), jnp.float32),
        mesh=mesh,
        scratch_types=[
            pltpu.VMEM((_SPW,), jnp.int32),
            pltpu.VMEM((_SPW,), jnp.int32),
            pltpu.VMEM((_CHUNK * _F,), jnp.float32),
            pltpu.VMEM((_CHUNK * _F,), jnp.float32),
            pltpu.VMEM((_SPW, _ND), jnp.float32),
            pltpu.SemaphoreType.DMA,
            pltpu.SemaphoreType.DMA,
        ],
        interpret=interpret,
    )
    def pool(x_hbm, s_hbm, e_hbm, out_hbm,
             svec_v, evec_v, buf0_v, buf1_v, stage_v, sem0, sem1):
        wid = lax.axis_index("s") * _NC + lax.axis_index("c")
        seg0 = wid * _SPW
        pltpu.sync_copy(s_hbm.at[pl.ds(seg0, _SPW)], svec_v)
        pltpu.sync_copy(e_hbm.at[pl.ds(seg0, _SPW)], evec_v)
        svec = svec_v[...]
        evec = evec_v[...]
        bounds = [svec[0]] + [evec[j] for j in range(_SPW)]

        zeros = jnp.zeros((_LANES,), jnp.float32)
        ninf = jnp.full((_LANES,), -jnp.inf, jnp.float32)
        for j in range(_SPW):
            for k in range(_F // _LANES):
                stage_v[j, pl.ds(k * _LANES, _LANES)] = zeros
                stage_v[j, pl.ds(_F + k * _LANES, _LANES)] = ninf

        w_lo = bounds[0]
        w_hi = bounds[_SPW]
        nrows = w_hi - w_lo
        nchunks = lax.div(nrows + (_CHUNK - 1), _CHUNK)

        def dma(c, buf, sem):
            base0 = w_lo + c * _CHUNK
            base = jnp.minimum(base0, V - _CHUNK)
            return pltpu.async_copy(
                x_hbm.at[pl.ds(base * _F, _CHUNK * _F)], buf, sem)

        def wait(buf, sem):
            pltpu.make_async_copy(
                x_hbm.at[pl.ds(0, _CHUNK * _F)], buf, sem).wait()

        def scalar_select(jj, vals):
            v = vals[0]
            for k in range(1, len(vals)):
                v = jnp.where(jj == k, vals[k], v)
            return v

        def process(c, buf, carry):
            base0 = w_lo + c * _CHUNK
            off = base0 - jnp.minimum(base0, V - _CHUNK)
            hi = jnp.clip(w_hi - base0, 0, _CHUNK)
            j, (ss, mm) = carry
            chunk_end = base0 + hi
            ended = jnp.int32(0)
            for jj in range(_SPW):
                ended = ended + (bounds[jj + 1] <= chunk_end).astype(jnp.int32)

            def row_body(rr, rc):
                rs, rm = rc
                ns, nm = [], []
                for k in range(_F // _LANES):
                    v = buf[pl.ds(rr * _F + k * _LANES, _LANES)]
                    ns.append(rs[k] + v)
                    nm.append(jnp.maximum(rm[k], v))
                return tuple(ns), tuple(nm)

            def run_rows(lo, hi_r, acc):
                nq = lax.div(hi_r - lo, 4)

                def quad(qp, a):
                    base = lo + qp * 4
                    for t in range(4):
                        a = row_body(base + t, a)
                    return a

                acc = lax.fori_loop(0, nq, quad, acc)
                return lax.fori_loop(lo + nq * 4, hi_r, row_body, acc)

            def flush_body(jj, st):
                r, fss, fmm = st
                sj = scalar_select(jj, bounds[:_SPW])
                ej = scalar_select(jj, bounds[1:])
                stop = jnp.clip(ej - base0, 0, hi)
                fss, fmm = run_rows(off + r, off + stop, (fss, fmm))
                nv = jnp.zeros((_LANES,), jnp.float32) + (ej - sj).astype(jnp.float32)
                inv = 1.0 / jnp.maximum(nv, 1.0)
                for k in range(_F // _LANES):
                    stage_v[jj, pl.ds(k * _LANES, _LANES)] = fss[k] * inv
                    stage_v[jj, pl.ds(_F + k * _LANES, _LANES)] = fmm[k]
                fss = tuple(jnp.zeros((_LANES,), jnp.float32)
                            for _ in range(_F // _LANES))
                fmm = tuple(jnp.full((_LANES,), -jnp.inf, jnp.float32)
                            for _ in range(_F // _LANES))
                return stop, fss, fmm

            r, ss, mm = lax.fori_loop(j, ended, flush_body,
                                      (jnp.int32(0), ss, mm))
            r = jnp.clip(r, 0, hi)
            ss, mm = run_rows(off + r, off + hi, (ss, mm))
            return (ended, (ss, mm))

        init = (
            jnp.int32(0),
            (
                tuple(jnp.zeros((_LANES,), jnp.float32)
                      for _ in range(_F // _LANES)),
                tuple(jnp.full((_LANES,), -jnp.inf, jnp.float32)
                      for _ in range(_F // _LANES)),
            ),
        )

        @pl.when(nchunks > 0)
        def _():
            dma(0, buf0_v, sem0)

        npairs = lax.div(nchunks + 1, 2)

        def pair_body(p, carry):
            c0 = 2 * p
            c1 = c0 + 1
            c2 = c0 + 2

            @pl.when(c1 < nchunks)
            def _():
                dma(c1, buf1_v, sem1)

            wait(buf0_v, sem0)
            carry = process(c0, buf0_v, carry)

            @pl.when(c2 < nchunks)
            def _():
                dma(c2, buf0_v, sem0)

            @pl.when(c1 < nchunks)
            def _():
                wait(buf1_v, sem1)

            carry = process(c1, buf1_v, carry)
            return carry

        lax.fori_loop(0, npairs, pair_body, init)

        pltpu.sync_copy(stage_v, out_hbm.at[pl.ds(seg0, _SPW), :])

    return pool


def _mlp_body(pr_ref, w1_ref, b1_ref, w2_ref, b2_ref, o_ref):
    pooled = pr_ref[...]
    h = lax.dot_general(pooled, w1_ref[...], (((1,), (1,)), ((), ())),
                        preferred_element_type=jnp.float32) + b1_ref[...]
    h = jnp.maximum(h, 0.0)
    o_ref[...] = lax.dot_general(h, w2_ref[...], (((1,), (1,)), ((), ())),
                                 preferred_element_type=jnp.float32) + b2_ref[...]


def _pooled_to_out(pr, W1, b1, W2, b2, interpret=False):
    return pl.pallas_call(
        _mlp_body,
        out_shape=jax.ShapeDtypeStruct((_G, _ND), jnp.float32),
        interpret=interpret,
    )(pr, W1, b1.reshape(1, _ND), W2, b2.reshape(1, _ND))


def kernel(x, node2graph, W1, b1, W2, b2):
    V = x.shape[0]
    ids = node2graph.astype(jnp.int32)
    gids = jnp.arange(_G, dtype=jnp.int32)
    stride = 32
    vpad = ((V + stride - 1) // stride) * stride
    ids_p = jnp.pad(ids, (0, vpad - V), constant_values=_G)
    sub = ids_p[::stride]
    coarse = jnp.sum(sub[:, None] <= gids[None, :], axis=0,
                     dtype=jnp.int32)
    ws = jnp.maximum(coarse - 1, 0) * stride
    win = jnp.take(ids_p, ws[:, None] + jnp.arange(stride, dtype=jnp.int32)[None, :])
    seg_end = ws + jnp.sum(win <= gids[:, None], axis=1, dtype=jnp.int32)
    seg_start = jnp.concatenate(
        [jnp.zeros((1,), jnp.int32), seg_end[:-1]])
    pooled = _build_pool(V)(x.reshape(-1), seg_start, seg_end)
    return _pooled_to_out(pooled, W1, b1, W2, b2)


# speedup vs baseline: 1.0023x; 1.0023x over previous
import functools

import jax
import jax.numpy as jnp
from jax import lax
from jax.experimental import pallas as pl
from jax.experimental.pallas import tpu as pltpu
from jax.experimental.pallas import tpu_sc as plsc

_G = 512
_F = 128
_ND = 2 * _F
_NC = 2
_NS = 16
_NW = _NC * _NS
_SPW = _G // _NW
_CHUNK = 384
_LANES = 16
_WIN = 128
_NB = _SPW + 1


def _build_pool(V, interpret=False):
    mesh = plsc.VectorSubcoreMesh(core_axis_name="c", subcore_axis_name="s",
                                  num_cores=_NC, num_subcores=_NS)

    @functools.partial(
        pl.kernel,
        out_type=jax.ShapeDtypeStruct((_G, _ND), jnp.float32),
        mesh=mesh,
        scratch_types=[
            pltpu.VMEM((_SPW,), jnp.int32),
            pltpu.VMEM((_SPW,), jnp.int32),
            pltpu.VMEM((_CHUNK * _F,), jnp.float32),
            pltpu.VMEM((_CHUNK * _F,), jnp.float32),
            pltpu.VMEM((_SPW, _ND), jnp.float32),
            pltpu.SemaphoreType.DMA,
            pltpu.SemaphoreType.DMA,
        ],
        interpret=interpret,
    )
    def pool(x_hbm, s_hbm, e_hbm, out_hbm,
             svec_v, evec_v, buf0_v, buf1_v, stage_v, sem0, sem1):
        wid = lax.axis_index("s") * _NC + lax.axis_index("c")
        seg0 = wid * _SPW
        pltpu.sync_copy(s_hbm.at[pl.ds(seg0, _SPW)], svec_v)
        pltpu.sync_copy(e_hbm.at[pl.ds(seg0, _SPW)], evec_v)
        svec = svec_v[...]
        evec = evec_v[...]
        bounds = [svec[0]] + [evec[j] for j in range(_SPW)]

        zeros = jnp.zeros((_LANES,), jnp.float32)
        ninf = jnp.full((_LANES,), -jnp.inf, jnp.float32)
        for j in range(_SPW):
            for k in range(_F // _LANES):
                stage_v[j, pl.ds(k * _LANES, _LANES)] = zeros
                stage_v[j, pl.ds(_F + k * _LANES, _LANES)] = ninf

        w_lo = bounds[0]
        w_hi = bounds[_SPW]
        nrows = w_hi - w_lo
        nchunks = lax.div(nrows + (_CHUNK - 1), _CHUNK)

        def dma(c, buf, sem):
            base0 = w_lo + c * _CHUNK
            base = jnp.minimum(base0, V - _CHUNK)
            return pltpu.async_copy(
                x_hbm.at[pl.ds(base * _F, _CHUNK * _F)], buf, sem)

        def wait(buf, sem):
            pltpu.make_async_copy(
                x_hbm.at[pl.ds(0, _CHUNK * _F)], buf, sem).wait()

        def scalar_select(jj, vals):
            v = vals[0]
            for k in range(1, len(vals)):
                v = jnp.where(jj == k, vals[k], v)
            return v

        def process(c, buf, carry):
            base0 = w_lo + c * _CHUNK
            off = base0 - jnp.minimum(base0, V - _CHUNK)
            hi = jnp.clip(w_hi - base0, 0, _CHUNK)
            j, (ss, mm) = carry
            chunk_end = base0 + hi
            ended = jnp.int32(0)
            for jj in range(_SPW):
                ended = ended + (bounds[jj + 1] <= chunk_end).astype(jnp.int32)

            def row_body(rr, rc):
                rs, rm = rc
                ns, nm = [], []
                for k in range(_F // _LANES):
                    v = buf[pl.ds(rr * _F + k * _LANES, _LANES)]
                    ns.append(rs[k] + v)
                    nm.append(jnp.maximum(rm[k], v))
                return tuple(ns), tuple(nm)

            def run_rows(lo, hi_r, acc):
                nq = lax.div(hi_r - lo, 4)

                def quad(qp, a):
                    base = lo + qp * 4
                    for t in range(4):
                        a = row_body(base + t, a)
                    return a

                acc = lax.fori_loop(0, nq, quad, acc)
                return lax.fori_loop(lo + nq * 4, hi_r, row_body, acc)

            def flush_body(jj, st):
                r, fss, fmm = st
                sj = scalar_select(jj, bounds[:_SPW])
                ej = scalar_select(jj, bounds[1:])
                stop = jnp.clip(ej - base0, 0, hi)
                fss, fmm = run_rows(off + r, off + stop, (fss, fmm))
                nv = jnp.zeros((_LANES,), jnp.float32) + (ej - sj).astype(jnp.float32)
                inv = 1.0 / jnp.maximum(nv, 1.0)
                for k in range(_F // _LANES):
                    stage_v[jj, pl.ds(k * _LANES, _LANES)] = fss[k] * inv
                    stage_v[jj, pl.ds(_F + k * _LANES, _LANES)] = fmm[k]
                fss = tuple(jnp.zeros((_LANES,), jnp.float32)
                            for _ in range(_F // _LANES))
                fmm = tuple(jnp.full((_LANES,), -jnp.inf, jnp.float32)
                            for _ in range(_F // _LANES))
                return stop, fss, fmm

            r, ss, mm = lax.fori_loop(j, ended, flush_body,
                                      (jnp.int32(0), ss, mm))
            r = jnp.clip(r, 0, hi)
            ss, mm = run_rows(off + r, off + hi, (ss, mm))
            return (ended, (ss, mm))

        init = (
            jnp.int32(0),
            (
                tuple(jnp.zeros((_LANES,), jnp.float32)
                      for _ in range(_F // _LANES)),
                tuple(jnp.full((_LANES,), -jnp.inf, jnp.float32)
                      for _ in range(_F // _LANES)),
            ),
        )

        @pl.when(nchunks > 0)
        def _():
            dma(0, buf0_v, sem0)

        npairs = lax.div(nchunks + 1, 2)

        def pair_body(p, carry):
            c0 = 2 * p
            c1 = c0 + 1
            c2 = c0 + 2

            @pl.when(c1 < nchunks)
            def _():
                dma(c1, buf1_v, sem1)

            wait(buf0_v, sem0)
            carry = process(c0, buf0_v, carry)

            @pl.when(c2 < nchunks)
            def _():
                dma(c2, buf0_v, sem0)

            @pl.when(c1 < nchunks)
            def _():
                wait(buf1_v, sem1)

            carry = process(c1, buf1_v, carry)
            return carry

        lax.fori_loop(0, npairs, pair_body, init)

        pltpu.sync_copy(stage_v, out_hbm.at[pl.ds(seg0, _SPW), :])

    return pool


def _mlp_body(pr_ref, w1_ref, b1_ref, w2_ref, b2_ref, o_ref):
    pooled = pr_ref[...]
    h = lax.dot_general(pooled, w1_ref[...], (((1,), (1,)), ((), ())),
                        preferred_element_type=jnp.float32) + b1_ref[...]
    h = jnp.maximum(h, 0.0)
    o_ref[...] = lax.dot_general(h, w2_ref[...], (((1,), (1,)), ((), ())),
                                 preferred_element_type=jnp.float32) + b2_ref[...]


def _pooled_to_out(pr, W1, b1, W2, b2, interpret=False):
    return pl.pallas_call(
        _mlp_body,
        out_shape=jax.ShapeDtypeStruct((_G, _ND), jnp.float32),
        interpret=interpret,
    )(pr, W1, b1.reshape(1, _ND), W2, b2.reshape(1, _ND))


def kernel(x, node2graph, W1, b1, W2, b2):
    V = x.shape[0]
    ids = node2graph.astype(jnp.int32)
    gids = jnp.arange(_G, dtype=jnp.int32)
    stride = 32
    vpad = ((V + stride - 1) // stride) * stride
    ids_p = jnp.pad(ids, (0, vpad - V), constant_values=_G)
    sub = ids_p[::stride]
    coarse = jnp.sum(sub[:, None] <= gids[None, :], axis=0,
                     dtype=jnp.int32)
    ws = jnp.maximum(coarse - 1, 0) * stride
    win = jnp.take(ids_p, ws[:, None] + jnp.arange(stride, dtype=jnp.int32)[None, :])
    seg_end = ws + jnp.sum(win <= gids[:, None], axis=1, dtype=jnp.int32)
    seg_start = jnp.concatenate(
        [jnp.zeros((1,), jnp.int32), seg_end[:-1]])
    pooled = _build_pool(V)(x.reshape(-1), seg_start, seg_end)
    return _pooled_to_out(pooled, W1, b1, W2, b2)


# speedup vs baseline: 1.0122x; 1.0099x over previous
import functools

import jax
import jax.numpy as jnp
from jax import lax
from jax.experimental import pallas as pl
from jax.experimental.pallas import tpu as pltpu
from jax.experimental.pallas import tpu_sc as plsc

_G = 512
_F = 128
_ND = 2 * _F
_NC = 2
_NS = 16
_NW = _NC * _NS
_SPW = _G // _NW
_CHUNK = 256
_LANES = 16
_WIN = 128
_NB = _SPW + 1


def _build_pool(V, interpret=False):
    mesh = plsc.VectorSubcoreMesh(core_axis_name="c", subcore_axis_name="s",
                                  num_cores=_NC, num_subcores=_NS)

    @functools.partial(
        pl.kernel,
        out_type=jax.ShapeDtypeStruct((_G, _ND), jnp.float32),
        mesh=mesh,
        scratch_types=[
            pltpu.VMEM((_SPW,), jnp.int32),
            pltpu.VMEM((_SPW,), jnp.int32),
            pltpu.VMEM((_CHUNK * _F,), jnp.float32),
            pltpu.VMEM((_CHUNK * _F,), jnp.float32),
            pltpu.VMEM((_SPW, _ND), jnp.float32),
            pltpu.SemaphoreType.DMA,
            pltpu.SemaphoreType.DMA,
        ],
        interpret=interpret,
    )
    def pool(x_hbm, s_hbm, e_hbm, out_hbm,
             svec_v, evec_v, buf0_v, buf1_v, stage_v, sem0, sem1):
        wid = lax.axis_index("s") * _NC + lax.axis_index("c")
        seg0 = wid * _SPW
        pltpu.sync_copy(s_hbm.at[pl.ds(seg0, _SPW)], svec_v)
        pltpu.sync_copy(e_hbm.at[pl.ds(seg0, _SPW)], evec_v)
        svec = svec_v[...]
        evec = evec_v[...]
        bounds = [svec[0]] + [evec[j] for j in range(_SPW)]

        zeros = jnp.zeros((_LANES,), jnp.float32)
        ninf = jnp.full((_LANES,), -jnp.inf, jnp.float32)
        for j in range(_SPW):
            for k in range(_F // _LANES):
                stage_v[j, pl.ds(k * _LANES, _LANES)] = zeros
                stage_v[j, pl.ds(_F + k * _LANES, _LANES)] = ninf

        w_lo = bounds[0]
        w_hi = bounds[_SPW]
        nrows = w_hi - w_lo
        nchunks = lax.div(nrows + (_CHUNK - 1), _CHUNK)

        def dma(c, buf, sem):
            base0 = w_lo + c * _CHUNK
            base = jnp.minimum(base0, V - _CHUNK)
            return pltpu.async_copy(
                x_hbm.at[pl.ds(base * _F, _CHUNK * _F)], buf, sem)

        def wait(buf, sem):
            pltpu.make_async_copy(
                x_hbm.at[pl.ds(0, _CHUNK * _F)], buf, sem).wait()

        def scalar_select(jj, vals):
            v = vals[0]
            for k in range(1, len(vals)):
                v = jnp.where(jj == k, vals[k], v)
            return v

        def process(c, buf, carry):
            base0 = w_lo + c * _CHUNK
            off = base0 - jnp.minimum(base0, V - _CHUNK)
            hi = jnp.clip(w_hi - base0, 0, _CHUNK)
            j, (ss, mm) = carry
            chunk_end = base0 + hi
            ended = jnp.int32(0)
            for jj in range(_SPW):
                ended = ended + (bounds[jj + 1] <= chunk_end).astype(jnp.int32)

            def row_body(rr, rc):
                rs, rm = rc
                ns, nm = [], []
                for k in range(_F // _LANES):
                    v = buf[pl.ds(rr * _F + k * _LANES, _LANES)]
                    ns.append(rs[k] + v)
                    nm.append(jnp.maximum(rm[k], v))
                return tuple(ns), tuple(nm)

            def run_rows(lo, hi_r, acc):
                return lax.fori_loop(lo, hi_r, row_body, acc)

            def flush_body(jj, st):
                r, fss, fmm = st
                sj = scalar_select(jj, bounds[:_SPW])
                ej = scalar_select(jj, bounds[1:])
                stop = jnp.clip(ej - base0, 0, hi)
                fss, fmm = run_rows(off + r, off + stop, (fss, fmm))
                nv = jnp.zeros((_LANES,), jnp.float32) + (ej - sj).astype(jnp.float32)
                inv = 1.0 / jnp.maximum(nv, 1.0)
                for k in range(_F // _LANES):
                    stage_v[jj, pl.ds(k * _LANES, _LANES)] = fss[k] * inv
                    stage_v[jj, pl.ds(_F + k * _LANES, _LANES)] = fmm[k]
                fss = tuple(jnp.zeros((_LANES,), jnp.float32)
                            for _ in range(_F // _LANES))
                fmm = tuple(jnp.full((_LANES,), -jnp.inf, jnp.float32)
                            for _ in range(_F // _LANES))
                return stop, fss, fmm

            r, ss, mm = lax.fori_loop(j, ended, flush_body,
                                      (jnp.int32(0), ss, mm))
            r = jnp.clip(r, 0, hi)
            ss, mm = run_rows(off + r, off + hi, (ss, mm))
            return (ended, (ss, mm))

        init = (
            jnp.int32(0),
            (
                tuple(jnp.zeros((_LANES,), jnp.float32)
                      for _ in range(_F // _LANES)),
                tuple(jnp.full((_LANES,), -jnp.inf, jnp.float32)
                      for _ in range(_F // _LANES)),
            ),
        )

        @pl.when(nchunks > 0)
        def _():
            dma(0, buf0_v, sem0)

        npairs = lax.div(nchunks + 1, 2)

        def pair_body(p, carry):
            c0 = 2 * p
            c1 = c0 + 1
            c2 = c0 + 2

            @pl.when(c1 < nchunks)
            def _():
                dma(c1, buf1_v, sem1)

            wait(buf0_v, sem0)
            carry = process(c0, buf0_v, carry)

            @pl.when(c2 < nchunks)
            def _():
                dma(c2, buf0_v, sem0)

            @pl.when(c1 < nchunks)
            def _():
                wait(buf1_v, sem1)

            carry = process(c1, buf1_v, carry)
            return carry

        lax.fori_loop(0, npairs, pair_body, init)

        pltpu.sync_copy(stage_v, out_hbm.at[pl.ds(seg0, _SPW), :])

    return pool


def _mlp_body(pr_ref, w1_ref, b1_ref, w2_ref, b2_ref, o_ref):
    pooled = pr_ref[...]
    h = lax.dot_general(pooled, w1_ref[...], (((1,), (1,)), ((), ())),
                        preferred_element_type=jnp.float32) + b1_ref[...]
    h = jnp.maximum(h, 0.0)
    o_ref[...] = lax.dot_general(h, w2_ref[...], (((1,), (1,)), ((), ())),
                                 preferred_element_type=jnp.float32) + b2_ref[...]


def _pooled_to_out(pr, W1, b1, W2, b2, interpret=False):
    return pl.pallas_call(
        _mlp_body,
        out_shape=jax.ShapeDtypeStruct((_G, _ND), jnp.float32),
        interpret=interpret,
    )(pr, W1, b1.reshape(1, _ND), W2, b2.reshape(1, _ND))


def kernel(x, node2graph, W1, b1, W2, b2):
    V = x.shape[0]
    ids = node2graph.astype(jnp.int32)
    gids = jnp.arange(_G, dtype=jnp.int32)
    stride = 32
    vpad = ((V + stride - 1) // stride) * stride
    ids_p = jnp.pad(ids, (0, vpad - V), constant_values=_G)
    sub = ids_p[::stride]
    coarse = jnp.sum(sub[:, None] <= gids[None, :], axis=0,
                     dtype=jnp.int32)
    ws = jnp.maximum(coarse - 1, 0) * stride
    win = jnp.take(ids_p, ws[:, None] + jnp.arange(stride, dtype=jnp.int32)[None, :])
    seg_end = ws + jnp.sum(win <= gids[:, None], axis=1, dtype=jnp.int32)
    seg_start = jnp.concatenate(
        [jnp.zeros((1,), jnp.int32), seg_end[:-1]])
    pooled = _build_pool(V)(x.reshape(-1), seg_start, seg_end)
    return _pooled_to_out(pooled, W1, b1, W2, b2)
